# baseline (speedup 1.0000x reference)
import functools

import jax
import jax.numpy as jnp
from jax import lax
from jax.experimental import pallas as pl

NC = 32
C = 256
D = 16
K = 16
ALPHA = 8
KB = 8
HS = 32
HM = 32
HMOD = 32
MOD_IN = K + 3 * D + 1
MOD_OUT = K + KB + 1 + D


def _cell_body(x_ref, h_ref, conn_ref, gate_ref, prev_ref, rest_ref,
               m1_ref, mb1_ref, m2_ref, mb2_ref,
               sw1_ref, sb1_ref, sw2_ref, sb2_ref,
               mw1_ref, mb1s_ref, mw2_ref, mb2s_ref,
               out_ref, *, bs):
    f32 = jnp.float32
    h_c = h_ref[...].reshape(bs, C, D)
    x_c = x_ref[...].reshape(bs, ALPHA, D)
    h_inj = jnp.concatenate([h_c[:, :ALPHA, :] + x_c, h_c[:, ALPHA:, :]],
                            axis=1)

    idx = conn_ref[...].reshape(ALPHA * K, 1)
    onehot = (idx == lax.broadcasted_iota(jnp.int32, (ALPHA * K, C), 1)
              ).astype(f32)
    gate = jax.nn.sigmoid(gate_ref[...].reshape(bs, ALPHA, K))
    gath_list = []
    for b in range(bs):
        rows = jnp.dot(onehot, h_inj[b], preferred_element_type=f32)
        rows = rows.reshape(ALPHA, K, D)
        gath_list.append((gate[b][:, :, None] * rows).sum(axis=1))
    gath = jnp.stack(gath_list, axis=0)

    h_r = h_c[:, C - ALPHA:, :]
    prev = prev_ref[...].reshape(bs, ALPHA, D)

    msg_inp = jnp.concatenate([h_r, gath, prev], axis=-1)
    flat = msg_inp.reshape(bs * ALPHA, 3 * D)
    mh = jnp.tanh(
        lax.dot_general(flat, mw1_ref[...], (((1,), (1,)), ((), ())),
                        preferred_element_type=f32) + mb1s_ref[...])
    msg = (lax.dot_general(mh, mw2_ref[...], (((1,), (1,)), ((), ())),
                           preferred_element_type=f32) + mb2s_ref[...])
    msg = msg.reshape(bs, ALPHA, D)

    rest = rest_ref[...].reshape(bs, ALPHA, MOD_IN - D)
    mod_inp = jnp.concatenate([rest[..., :K], h_r, rest[..., K:]], axis=-1)
    out_list = []
    for r in range(ALPHA):
        w1_r = m1_ref[r]
        b1_r = mb1_ref[r]
        w2_r = m2_ref[r]
        b2_r = mb2_ref[r]
        hid = jnp.tanh(
            lax.dot_general(mod_inp[:, r, :], w1_r, (((1,), (1,)), ((), ())),
                            preferred_element_type=f32) + b1_r)
        out_list.append(jnp.dot(hid, w2_r, preferred_element_type=f32) + b2_r)
    outm = jnp.stack(out_list, axis=1)

    nd = outm[:, :, K + KB:K + KB + 1]
    new_prim = outm[:, :, K + KB + 1:]

    st_inp = jnp.concatenate([h_r, msg, new_prim, nd], axis=-1)
    sflat = st_inp.reshape(bs * ALPHA, 3 * D + 1)
    sh = jnp.tanh(
        lax.dot_general(sflat, sw1_ref[...], (((1,), (1,)), ((), ())),
                        preferred_element_type=f32) + sb1_ref[...])
    delta = (lax.dot_general(sh, sw2_ref[...], (((1,), (1,)), ((), ())),
                             preferred_element_type=f32) + sb2_ref[...])
    delta = delta.reshape(bs, ALPHA, D)

    h_new = h_r * jax.nn.sigmoid(nd) + delta
    out_ref[...] = h_new.reshape(bs, 1, ALPHA, D)


def kernel(x, h, prev_messages, w_conn, decay_logit, primitives_state,
           hebbian_traces, state_w1, state_b1, state_w2, state_b2,
           msg_w1, msg_b1, msg_w2, msg_b2,
           mod_w1, mod_b1, mod_w2, mod_b2,
           neuron_id, conn_indices, border_indices):
    bs = x.shape[0]
    R = C - ALPHA

    conn_r = conn_indices[:, R:, :].reshape(NC, ALPHA * K, 1)
    gate_r = w_conn[:, :, R:, :]
    prev_r = prev_messages[:, :, R:, :]
    hebb_r = hebbian_traces[:, :, R:, :]
    decay_r = decay_logit[:, :, R:]
    prim_r = primitives_state[:, :, R:, :]
    nid_r = jnp.broadcast_to(neuron_id[None, :, R:, :], (bs, NC, ALPHA, D))
    rest = jnp.concatenate(
        [hebb_r, decay_r[..., None], prim_r, nid_r], axis=-1)

    grid = (NC,)
    body = functools.partial(_cell_body, bs=bs)
    out = pl.pallas_call(
        body,
        grid=grid,
        in_specs=[
            pl.BlockSpec((bs, 1, ALPHA, D), lambda i: (0, i, 0, 0)),
            pl.BlockSpec((bs, 1, C, D), lambda i: (0, i, 0, 0)),
            pl.BlockSpec((1, ALPHA * K, 1), lambda i: (i, 0, 0)),
            pl.BlockSpec((bs, 1, ALPHA, K), lambda i: (0, i, 0, 0)),
            pl.BlockSpec((bs, 1, ALPHA, D), lambda i: (0, i, 0, 0)),
            pl.BlockSpec((bs, 1, ALPHA, MOD_IN - D),
                         lambda i: (0, i, 0, 0)),
            pl.BlockSpec((ALPHA, HMOD, MOD_IN),
                         lambda i: (i * (C // ALPHA) + C // ALPHA - 1, 0, 0)),
            pl.BlockSpec((ALPHA, HMOD),
                         lambda i: (i * (C // ALPHA) + C // ALPHA - 1, 0)),
            pl.BlockSpec((ALPHA, HMOD, MOD_OUT),
                         lambda i: (i * (C // ALPHA) + C // ALPHA - 1, 0, 0)),
            pl.BlockSpec((ALPHA, MOD_OUT),
                         lambda i: (i * (C // ALPHA) + C // ALPHA - 1, 0)),
            pl.BlockSpec(state_w1.shape, lambda i: (0, 0)),
            pl.BlockSpec(state_b1.shape, lambda i: (0,)),
            pl.BlockSpec(state_w2.shape, lambda i: (0, 0)),
            pl.BlockSpec(state_b2.shape, lambda i: (0,)),
            pl.BlockSpec(msg_w1.shape, lambda i: (0, 0)),
            pl.BlockSpec(msg_b1.shape, lambda i: (0,)),
            pl.BlockSpec(msg_w2.shape, lambda i: (0, 0)),
            pl.BlockSpec(msg_b2.shape, lambda i: (0,)),
        ],
        out_specs=pl.BlockSpec((bs, 1, ALPHA, D), lambda i: (0, i, 0, 0)),
        out_shape=jax.ShapeDtypeStruct((bs, NC, ALPHA, D), jnp.float32),
    )(x, h, conn_r, gate_r, prev_r, rest, mod_w1, mod_b1, mod_w2, mod_b2,
      state_w1, state_b1, state_w2, state_b2,
      msg_w1, msg_b1, msg_w2, msg_b2)
    return out


# speedup vs baseline: 1.5676x; 1.5676x over previous
import functools

import jax
import jax.numpy as jnp
from jax import lax
from jax.experimental import pallas as pl

NC = 32
C = 256
D = 16
K = 16
ALPHA = 8
KB = 8
HS = 32
HM = 32
HMOD = 32
MOD_IN = K + 3 * D + 1
MOD_OUT = K + KB + 1 + D
CPB = 8


def _body(x_ref, h_ref, conn_ref, gate_ref, prev_ref, rest_ref,
          m1_ref, mb1_ref, m2_ref, mb2_ref,
          sw1_ref, sb1_ref, sw2_ref, sb2_ref,
          mw1_ref, mb1s_ref, mw2_ref, mb2s_ref,
          out_ref, *, bs):
    f32 = jnp.float32
    h_all = h_ref[...]
    x_all = x_ref[...]
    gate = jax.nn.sigmoid(gate_ref[...])
    ciota = lax.broadcasted_iota(jnp.int32, (ALPHA * K, C), 1)

    gath_cells = []
    for ci in range(CPB):
        h_c = h_all[:, ci]
        h_inj = jnp.concatenate(
            [h_c[:, :ALPHA, :] + x_all[:, ci], h_c[:, ALPHA:, :]], axis=1)
        idx = conn_ref[...][ci]
        onehot = (idx == ciota).astype(f32)
        g_list = []
        for b in range(bs):
            rows = jnp.dot(onehot, h_inj[b], preferred_element_type=f32)
            rows = rows.reshape(ALPHA, K, D)
            g_list.append((gate[b, ci][:, :, None] * rows).sum(axis=1))
        gath_cells.append(jnp.stack(g_list, axis=0))
    gath = jnp.stack(gath_cells, axis=1)

    h_r = h_all[:, :, C - ALPHA:, :]
    prev = prev_ref[...]

    msg_inp = jnp.concatenate([h_r, gath, prev], axis=-1)
    flat = msg_inp.reshape(bs * CPB * ALPHA, 3 * D)
    mh = jnp.tanh(
        lax.dot_general(flat, mw1_ref[...], (((1,), (1,)), ((), ())),
                        preferred_element_type=f32) + mb1s_ref[...])
    msg = (lax.dot_general(mh, mw2_ref[...], (((1,), (1,)), ((), ())),
                           preferred_element_type=f32) + mb2s_ref[...])
    msg = msg.reshape(bs, CPB, ALPHA, D)

    rest = rest_ref[...]
    mod_inp = jnp.concatenate(
        [rest[..., :K], h_r, rest[..., K:]], axis=-1)
    m1 = m1_ref[...].reshape(CPB, ALPHA, HMOD, MOD_IN)
    mb1 = mb1_ref[...].reshape(CPB, ALPHA, HMOD)
    m2 = m2_ref[...].reshape(CPB, ALPHA, HMOD, MOD_OUT)
    mb2 = mb2_ref[...].reshape(CPB, ALPHA, MOD_OUT)
    out_cells = []
    for ci in range(CPB):
        out_list = []
        for r in range(ALPHA):
            hid = jnp.tanh(
                lax.dot_general(mod_inp[:, ci, r, :], m1[ci, r],
                                (((1,), (1,)), ((), ())),
                                preferred_element_type=f32) + mb1[ci, r])
            out_list.append(
                jnp.dot(hid, m2[ci, r], preferred_element_type=f32)
                + mb2[ci, r])
        out_cells.append(jnp.stack(out_list, axis=1))
    outm = jnp.stack(out_cells, axis=1)

    nd = outm[..., K + KB:K + KB + 1]
    new_prim = outm[..., K + KB + 1:]

    st_inp = jnp.concatenate([h_r, msg, new_prim, nd], axis=-1)
    sflat = st_inp.reshape(bs * CPB * ALPHA, 3 * D + 1)
    sh = jnp.tanh(
        lax.dot_general(sflat, sw1_ref[...], (((1,), (1,)), ((), ())),
                        preferred_element_type=f32) + sb1_ref[...])
    delta = (lax.dot_general(sh, sw2_ref[...], (((1,), (1,)), ((), ())),
                             preferred_element_type=f32) + sb2_ref[...])
    delta = delta.reshape(bs, CPB, ALPHA, D)

    out_ref[...] = h_r * jax.nn.sigmoid(nd) + delta


def kernel(x, h, prev_messages, w_conn, decay_logit, primitives_state,
           hebbian_traces, state_w1, state_b1, state_w2, state_b2,
           msg_w1, msg_b1, msg_w2, msg_b2,
           mod_w1, mod_b1, mod_w2, mod_b2,
           neuron_id, conn_indices, border_indices):
    bs = x.shape[0]
    R = C - ALPHA
    G = C // ALPHA

    conn_r = conn_indices[:, R:, :].reshape(NC, ALPHA * K, 1)
    gate_r = w_conn[:, :, R:, :]
    prev_r = prev_messages[:, :, R:, :]
    hebb_r = hebbian_traces[:, :, R:, :]
    decay_r = decay_logit[:, :, R:]
    prim_r = primitives_state[:, :, R:, :]
    nid_r = jnp.broadcast_to(neuron_id[None, :, R:, :], (bs, NC, ALPHA, D))
    rest = jnp.concatenate(
        [hebb_r, decay_r[..., None], prim_r, nid_r], axis=-1)

    m1 = mod_w1.reshape(NC, G, ALPHA, HMOD, MOD_IN)
    mb1 = mod_b1.reshape(NC, G, ALPHA, HMOD)
    m2 = mod_w2.reshape(NC, G, ALPHA, HMOD, MOD_OUT)
    mb2 = mod_b2.reshape(NC, G, ALPHA, MOD_OUT)

    grid = (NC // CPB,)
    body = functools.partial(_body, bs=bs)
    out = pl.pallas_call(
        body,
        grid=grid,
        in_specs=[
            pl.BlockSpec((bs, CPB, ALPHA, D), lambda i: (0, i, 0, 0)),
            pl.BlockSpec((bs, CPB, C, D), lambda i: (0, i, 0, 0)),
            pl.BlockSpec((CPB, ALPHA * K, 1), lambda i: (i, 0, 0)),
            pl.BlockSpec((bs, CPB, ALPHA, K), lambda i: (0, i, 0, 0)),
            pl.BlockSpec((bs, CPB, ALPHA, D), lambda i: (0, i, 0, 0)),
            pl.BlockSpec((bs, CPB, ALPHA, MOD_IN - D),
                         lambda i: (0, i, 0, 0)),
            pl.BlockSpec((CPB, 1, ALPHA, HMOD, MOD_IN),
                         lambda i: (i, G - 1, 0, 0, 0)),
            pl.BlockSpec((CPB, 1, ALPHA, HMOD), lambda i: (i, G - 1, 0, 0)),
            pl.BlockSpec((CPB, 1, ALPHA, HMOD, MOD_OUT),
                         lambda i: (i, G - 1, 0, 0, 0)),
            pl.BlockSpec((CPB, 1, ALPHA, MOD_OUT), lambda i: (i, G - 1, 0, 0)),
            pl.BlockSpec(state_w1.shape, lambda i: (0, 0)),
            pl.BlockSpec(state_b1.shape, lambda i: (0,)),
            pl.BlockSpec(state_w2.shape, lambda i: (0, 0)),
            pl.BlockSpec(state_b2.shape, lambda i: (0,)),
            pl.BlockSpec(msg_w1.shape, lambda i: (0, 0)),
            pl.BlockSpec(msg_b1.shape, lambda i: (0,)),
            pl.BlockSpec(msg_w2.shape, lambda i: (0, 0)),
            pl.BlockSpec(msg_b2.shape, lambda i: (0,)),
        ],
        out_specs=pl.BlockSpec((bs, CPB, ALPHA, D), lambda i: (0, i, 0, 0)),
        out_shape=jax.ShapeDtypeStruct((bs, NC, ALPHA, D), jnp.float32),
    )(x, h, conn_r, gate_r, prev_r, rest, m1, mb1, m2, mb2,
      state_w1, state_b1, state_w2, state_b2,
      msg_w1, msg_b1, msg_w2, msg_b2)
    return out
